# baseline (speedup 1.0000x reference)
import functools

import jax
import jax.numpy as jnp
from jax import lax
from jax.experimental import pallas as pl
from jax.experimental.pallas import tpu as pltpu
from jax.experimental.pallas import tpu_sc as plsc

VOCAB = 1000000
EMBED = 64
HIDDEN = 64
B = 1024
T = 200

NUM_CORES = 2
NUM_SUBCORES = 16
NUM_WORKERS = NUM_CORES * NUM_SUBCORES

GATHER_CHUNK = 128

S_PER_BLOCK = 8


def _sc_gather(table, idx_flat):
    n = idx_flat.shape[0]
    per_w = n // NUM_WORKERS
    chunks = per_w // GATHER_CHUNK
    assert per_w * NUM_WORKERS == n and chunks * GATHER_CHUNK == per_w

    mesh = plsc.VectorSubcoreMesh(core_axis_name="c", subcore_axis_name="s")

    @functools.partial(
        pl.kernel,
        out_type=jax.ShapeDtypeStruct((n, EMBED), jnp.float32),
        mesh=mesh,
        scratch_types=[
            pltpu.VMEM((GATHER_CHUNK,), jnp.int32),
            pltpu.VMEM((GATHER_CHUNK, EMBED), jnp.float32),
            pltpu.SemaphoreType.DMA,
        ],
        compiler_params=pltpu.CompilerParams(use_tc_tiling_on_sc=False),
    )
    def gather_kernel(table_hbm, idx_hbm, out_hbm, idx_v, rows_v, sem):
        wid = lax.axis_index("s") * NUM_CORES + lax.axis_index("c")
        base_w = wid * per_w

        @pl.loop(0, chunks)
        def _chunk(c):
            base = base_w + c * GATHER_CHUNK
            pltpu.sync_copy(idx_hbm.at[pl.ds(base, GATHER_CHUNK)], idx_v)
            pltpu.async_copy(table_hbm.at[idx_v], rows_v, sem).wait()
            pltpu.sync_copy(rows_v, out_hbm.at[pl.ds(base, GATHER_CHUNK)])

    return gather_kernel(table, idx_flat)


def _lstm_body(xs_ref, wx_ref, wh_ref, b_ref, ys_ref, h_ref, c_ref):
    gi = pl.program_id(0)

    @pl.when(gi == 0)
    def _init():
        h_ref[...] = jnp.zeros_like(h_ref)
        c_ref[...] = jnp.zeros_like(c_ref)

    h = h_ref[...]
    c = c_ref[...]
    wx = wx_ref[...]
    wh = wh_ref[...]
    bias = b_ref[...]
    for k in range(S_PER_BLOCK):
        x = xs_ref[k * B:(k + 1) * B, :]
        gates = (jnp.dot(x, wx, preferred_element_type=jnp.float32)
                 + jnp.dot(h, wh, preferred_element_type=jnp.float32)
                 + bias)
        gi_ = jax.nn.sigmoid(gates[:, 0 * HIDDEN:1 * HIDDEN])
        gf = jax.nn.sigmoid(gates[:, 1 * HIDDEN:2 * HIDDEN])
        gg = jnp.tanh(gates[:, 2 * HIDDEN:3 * HIDDEN])
        go = jax.nn.sigmoid(gates[:, 3 * HIDDEN:4 * HIDDEN])
        c = gf * c + gi_ * gg
        h = go * jnp.tanh(c)
        ys_ref[:, k * HIDDEN:(k + 1) * HIDDEN] = h
    h_ref[...] = h
    c_ref[...] = c


def _tc_lstm(xs, wx, wh, bias):
    nblk = T // S_PER_BLOCK
    return pl.pallas_call(
        _lstm_body,
        grid=(nblk,),
        in_specs=[
            pl.BlockSpec((S_PER_BLOCK * B, EMBED), lambda i: (i, 0)),
            pl.BlockSpec((EMBED, 4 * HIDDEN), lambda i: (0, 0)),
            pl.BlockSpec((HIDDEN, 4 * HIDDEN), lambda i: (0, 0)),
            pl.BlockSpec((1, 4 * HIDDEN), lambda i: (0, 0)),
        ],
        out_specs=[
            pl.BlockSpec((B, S_PER_BLOCK * HIDDEN), lambda i: (0, i)),
            pl.BlockSpec((B, HIDDEN), lambda i: (0, 0)),
            pl.BlockSpec((B, HIDDEN), lambda i: (0, 0)),
        ],
        out_shape=[
            jax.ShapeDtypeStruct((B, T * HIDDEN), jnp.float32),
            jax.ShapeDtypeStruct((B, HIDDEN), jnp.float32),
            jax.ShapeDtypeStruct((B, HIDDEN), jnp.float32),
        ],
        compiler_params=pltpu.CompilerParams(
            dimension_semantics=("arbitrary",),
        ),
    )(xs, wx, wh, bias)


def kernel(src, table, W_ih, W_hh, b_ih, b_hh):
    idx_flat = jnp.transpose(src).reshape(-1)
    xs = _sc_gather(table, idx_flat)

    wx = jnp.transpose(W_ih)
    wh = jnp.transpose(W_hh)
    bias = (b_ih + b_hh)[None, :]

    ys, hT, cT = _tc_lstm(xs, wx, wh, bias)
    outputs = ys.reshape(B, T, HIDDEN)
    return (outputs, hT[None, :, :], cT[None, :, :])
